# Initial kernel scaffold; baseline (speedup 1.0000x reference)
import jax
import jax.numpy as jnp
from jax.experimental import pallas as pl


def kernel(input_ids, token_type_ids, token_embedding, segment_embedding, position_embedding):
    raise NotImplementedError("write your pallas kernel here")


# speedup vs baseline: 1.5716x; 1.5716x over previous
import functools

import jax
import jax.numpy as jnp
from jax import lax
from jax.experimental import pallas as pl
from jax.experimental.pallas import tpu as pltpu
from jax.experimental.pallas import tpu_sc as plsc

LANES = 16
NC, NS = 2, 16
NW = NC * NS
CHUNK = 64


def _posseg_body(seg_ref, pos_ref, out_ref):
    t = seg_ref.shape[0]
    s = pos_ref.shape[0]
    for i in range(t):
        out_ref[i * s:(i + 1) * s, :] = pos_ref[...] + seg_ref[i:i + 1, :]


def _make_posseg(seg, pos):
    t, d = seg.shape
    s = pos.shape[0]
    return pl.pallas_call(
        _posseg_body,
        out_shape=jax.ShapeDtypeStruct((t * s, d), jnp.float32),
    )(seg, pos)


def _sc_body(seq, total_rows, d,
             tok_hbm, posseg_hbm, ids_hbm, tt_hbm, out_hbm,
             idx_tok, idx_ps, ttv, tokbuf, psbuf, sem_t, sem_p):
    vecs = d // LANES
    rows_per_w = total_rows // NW
    n_chunks = rows_per_w // CHUNK
    blocks_per_seq = seq // CHUNK
    wid = lax.axis_index("s") * NC + lax.axis_index("c")
    wbase = wid * rows_per_w
    iota = lax.iota(jnp.int32, LANES)

    @pl.loop(0, n_chunks)
    def _chunk(g):
        base = wbase + g * CHUNK
        s0 = lax.rem(g, blocks_per_seq) * CHUNK
        pltpu.sync_copy(ids_hbm.at[pl.ds(base, CHUNK)], idx_tok)
        pltpu.sync_copy(tt_hbm.at[pl.ds(base, CHUNK)], ttv)
        for j in range(CHUNK // LANES):
            sl = pl.ds(j * LANES, LANES)
            idx_ps[sl] = ttv[sl] * seq + (s0 + j * LANES) + iota
        cp_t = pltpu.async_copy(tok_hbm.at[idx_tok], tokbuf, sem_t)
        cp_p = pltpu.async_copy(posseg_hbm.at[idx_ps], psbuf, sem_p)
        cp_t.wait()
        cp_p.wait()

        @pl.loop(0, CHUNK)
        def _row(i):
            for j in range(vecs):
                sl = pl.ds(j * LANES, LANES)
                tokbuf[i, sl] = tokbuf[i, sl] + psbuf[i, sl]

        pltpu.sync_copy(tokbuf, out_hbm.at[pl.ds(base, CHUNK)])


@functools.lru_cache(maxsize=None)
def _make_sc(seq, total_rows, d):
    assert total_rows % (NW * CHUNK) == 0
    assert seq % CHUNK == 0 and (total_rows // NW) % seq == 0
    assert d % LANES == 0
    mesh = plsc.VectorSubcoreMesh(
        core_axis_name="c", subcore_axis_name="s",
        num_cores=NC, num_subcores=NS)
    return pl.kernel(
        functools.partial(_sc_body, seq, total_rows, d),
        out_type=jax.ShapeDtypeStruct((total_rows, d), jnp.float32),
        mesh=mesh,
        scratch_types=[
            pltpu.VMEM((CHUNK,), jnp.int32),
            pltpu.VMEM((CHUNK,), jnp.int32),
            pltpu.VMEM((CHUNK,), jnp.int32),
            pltpu.VMEM((CHUNK, d), jnp.float32),
            pltpu.VMEM((CHUNK, d), jnp.float32),
            pltpu.SemaphoreType.DMA,
            pltpu.SemaphoreType.DMA,
        ],
    )


def kernel(input_ids, token_type_ids, token_embedding, segment_embedding,
           position_embedding):
    b, s = input_ids.shape
    d = token_embedding.shape[1]
    posseg = _make_posseg(segment_embedding, position_embedding)
    ids = input_ids.reshape(-1).astype(jnp.int32)
    tt = token_type_ids.reshape(-1).astype(jnp.int32)
    sc = _make_sc(s, b * s, d)
    out = sc(token_embedding, posseg, ids, tt)
    return out.reshape(b, s, d)


# speedup vs baseline: 2.3500x; 1.4953x over previous
import functools

import jax
import jax.numpy as jnp
from jax import lax
from jax.experimental import pallas as pl
from jax.experimental.pallas import tpu as pltpu
from jax.experimental.pallas import tpu_sc as plsc

LANES = 16
NC, NS = 2, 16
NW = NC * NS
CHUNK = 32
NSLOT = 4


def _pre_body(seg_ref, pos_ref, pos0_ref, delta_ref):
    pos0_ref[...] = pos_ref[...] + seg_ref[0:1, :]
    delta_ref[...] = seg_ref[1:2, :] - seg_ref[0:1, :]


def _make_pre(seg, pos):
    t, d = seg.shape
    s = pos.shape[0]
    assert t == 2
    return pl.pallas_call(
        _pre_body,
        out_shape=(jax.ShapeDtypeStruct((s, d), jnp.float32),
                   jax.ShapeDtypeStruct((1, d), jnp.float32)),
    )(seg, pos)


def _sc_body(seq, total_rows, d,
             tok_hbm, pos0_hbm, delta_hbm, ids_hbm, tt_hbm, out_hbm,
             idxc, ttc, pos0blk, delta_v, bufs, sem_g, sem_w):
    vecs = d // LANES
    rows_per_w = total_rows // NW
    seqs_per_w = rows_per_w // seq
    pblocks = seq // CHUNK
    k_iters = seqs_per_w // NSLOT
    wid = lax.axis_index("s") * NC + lax.axis_index("c")
    wbase = wid * rows_per_w
    iota = lax.iota(jnp.int32, LANES)
    zero16 = iota * 0

    pltpu.sync_copy(delta_hbm, delta_v)
    dvecs = [delta_v[0, pl.ds(j * LANES, LANES)] for j in range(vecs)]

    def row_base(q, p):
        return wbase + q * seq + p * CHUNK

    def issue(slot, q, p):
        base = row_base(q, p)
        pltpu.sync_copy(ids_hbm.at[pl.ds(base, CHUNK)], idxc.at[slot])
        pltpu.sync_copy(tt_hbm.at[pl.ds(base, CHUNK)],
                        ttc.at[pl.ds(slot * CHUNK, CHUNK)])
        pltpu.async_copy(tok_hbm.at[idxc.at[slot]], bufs.at[slot], sem_g[slot])

    def wait_gather(slot):
        pltpu.make_async_copy(
            tok_hbm.at[pl.ds(0, CHUNK)], bufs.at[slot], sem_g[slot]).wait()

    def splat(v, l):
        idx = (zero16 + l)[:, None]
        dn = lax.GatherDimensionNumbers(
            offset_dims=(), collapsed_slice_dims=(0,), start_index_map=(0,))
        return lax.gather(v, idx, dn, slice_sizes=(1,),
                          mode=lax.GatherScatterMode.PROMISE_IN_BOUNDS)

    def vadd_and_write(slot, q, p):
        buf = bufs.at[slot]

        @pl.loop(0, CHUNK // LANES)
        def _grp(g):
            tvec = ttc[pl.ds(slot * CHUNK + g * LANES, LANES)]
            tfv = tvec.astype(jnp.float32)

            @pl.loop(0, LANES)
            def _lane(l):
                i = g * LANES + l
                tf = splat(tfv, l)
                for j in range(vecs):
                    sl = pl.ds(j * LANES, LANES)
                    buf[i, sl] = buf[i, sl] + (pos0blk[i, sl] + tf * dvecs[j])

        base = row_base(q, p)
        pltpu.async_copy(buf, out_hbm.at[pl.ds(base, CHUNK)], sem_w[slot])

    def wait_write(slot):
        pltpu.make_async_copy(
            bufs.at[slot], out_hbm.at[pl.ds(0, CHUNK)], sem_w[slot]).wait()

    @pl.loop(0, pblocks)
    def _pblock(p):
        pltpu.sync_copy(pos0_hbm.at[pl.ds(p * CHUNK, CHUNK)], pos0blk)
        for s in range(NSLOT):
            issue(s, s, p)

        @pl.loop(0, k_iters)
        def _k(k):
            for s in range(NSLOT):
                wait_gather(s)
                vadd_and_write(s, k * NSLOT + s, p)

            @pl.when(k < k_iters - 1)
            def _reissue():
                for s in range(NSLOT):
                    wait_write(s)
                    issue(s, (k + 1) * NSLOT + s, p)

        for s in range(NSLOT):
            wait_write(s)


@functools.lru_cache(maxsize=None)
def _make_sc(seq, total_rows, d):
    rows_per_w = total_rows // NW
    assert total_rows % NW == 0 and rows_per_w % seq == 0
    assert seq % CHUNK == 0 and (rows_per_w // seq) % NSLOT == 0
    assert d % LANES == 0
    mesh = plsc.VectorSubcoreMesh(
        core_axis_name="c", subcore_axis_name="s",
        num_cores=NC, num_subcores=NS)
    return pl.kernel(
        functools.partial(_sc_body, seq, total_rows, d),
        out_type=jax.ShapeDtypeStruct((total_rows, d), jnp.float32),
        mesh=mesh,
        scratch_types=[
            pltpu.VMEM((NSLOT, CHUNK), jnp.int32),
            pltpu.VMEM((NSLOT * CHUNK,), jnp.int32),
            pltpu.VMEM((CHUNK, d), jnp.float32),
            pltpu.VMEM((1, d), jnp.float32),
            pltpu.VMEM((NSLOT, CHUNK, d), jnp.float32),
            [pltpu.SemaphoreType.DMA] * NSLOT,
            [pltpu.SemaphoreType.DMA] * NSLOT,
        ],
    )


def kernel(input_ids, token_type_ids, token_embedding, segment_embedding,
           position_embedding):
    b, s = input_ids.shape
    d = token_embedding.shape[1]
    pos0, delta = _make_pre(segment_embedding, position_embedding)
    ids = input_ids.reshape(-1).astype(jnp.int32)
    tt = token_type_ids.reshape(-1).astype(jnp.int32)
    sc = _make_sc(s, b * s, d)
    out = sc(token_embedding, pos0, delta, ids, tt)
    return out.reshape(b, s, d)
